# baseline (speedup 1.0000x reference)
import functools

import jax
import jax.numpy as jnp
from jax import lax
from jax.experimental import pallas as pl
from jax.experimental.pallas import tpu as pltpu
from jax.experimental.pallas import tpu_sc as plsc

_NUM_CLASSES = 100000
_SPECIAL_ROW = _NUM_CLASSES + 1
_N = 16384
_D = 64
_IDX_CHUNK = 128
_LANES = 16


@functools.lru_cache(maxsize=None)
def _make_lookup():
    info = plsc.get_sparse_core_info()
    nw = info.num_cores * info.num_subcores
    b_per_w = _N // nw
    n_chunks = b_per_w // _IDX_CHUNK
    mesh = plsc.VectorSubcoreMesh(core_axis_name="c", subcore_axis_name="s")

    @functools.partial(
        pl.kernel,
        mesh=mesh,
        out_type=jax.ShapeDtypeStruct((nw, b_per_w, _D), jnp.float32),
        scratch_types=[
            pltpu.VMEM((n_chunks, _IDX_CHUNK), jnp.int32),
            pltpu.VMEM((b_per_w, _D), jnp.float32),
            pltpu.SemaphoreType.DMA,
        ],
        compiler_params=pltpu.CompilerParams(use_tc_tiling_on_sc=False),
    )
    def lookup(labels_hbm, table_hbm, out_hbm, idx_v, rows_v, sem):
        wid = lax.axis_index("s") * info.num_cores + lax.axis_index("c")
        pltpu.sync_copy(labels_hbm.at[wid], idx_v)
        for j in range(n_chunks):
            for i in range(_IDX_CHUNK // _LANES):
                v = idx_v[j, pl.ds(i * _LANES, _LANES)]
                idx_v[j, pl.ds(i * _LANES, _LANES)] = jnp.where(
                    v < 0, _SPECIAL_ROW, v
                )
        copies = [
            pltpu.async_copy(
                table_hbm.at[idx_v.at[j]],
                rows_v.at[pl.ds(j * _IDX_CHUNK, _IDX_CHUNK)],
                sem,
            )
            for j in range(n_chunks)
        ]
        for c in copies:
            c.wait()
        pltpu.sync_copy(rows_v, out_hbm.at[wid])

    return lookup, nw, n_chunks


def kernel(labels, train, embedding_table):
    if labels.ndim == 0:
        labels = labels[None]
    lookup, nw, n_chunks = _make_lookup()
    labels3 = labels.reshape(nw, n_chunks, _IDX_CHUNK)
    out = lookup(labels3, embedding_table)
    return out.reshape(_N, _D)
